# baseline (speedup 1.0000x reference)
import jax
import jax.numpy as jnp
from jax import lax
from jax.experimental import pallas as pl
from jax.experimental.pallas import tpu as pltpu
from jax.experimental.pallas import tpu_sc as plsc

N = 10000
D = 128
E = 320000
NC, NS = 2, 16
NW = NC * NS
CH = 128
CPT = 80
EPT = CH * CPT
EH = EPT * NS
E_PAD = EH * NC
NROW = E_PAD // CH
N_ACC = 10240
SLAB = N_ACC // NS
ZR = 80


def _mesh():
    return plsc.VectorSubcoreMesh(core_axis_name="c", subcore_axis_name="s")


NHR = N_ACC // CH


def _hist_body(d0, d1, d2, out, deg0, deg1, deg2, ones_v, zv,
               dc0, dc1, dc2, dc3, dsem, ssem):
    cid = lax.axis_index("c")
    sid = lax.axis_index("s")
    vec1 = jnp.where(
        lax.broadcasted_iota(jnp.int32, (16,), 0) == 0,
        jnp.float32(1.0), jnp.float32(0.0))
    z16 = jnp.zeros((16,), jnp.float32)

    def init_ones(i, c):
        ones_v[i, :] = vec1
        return c
    lax.fori_loop(0, CH, init_ones, 0)

    def init_z(i, c):
        zv[i, :] = z16
        return c
    lax.fori_loop(0, SLAB, init_z, 0)

    row0 = sid * SLAB
    for dg in (deg0, deg1, deg2):
        pltpu.sync_copy(zv, dg.at[pl.ds(row0, SLAB), :])
    plsc.subcore_barrier()

    base = (cid * NS + sid) * EPT

    for dh, dg in ((d0, deg0), (d1, deg1), (d2, deg2)):
        def chunk(ci, c, dh=dh, dg=dg):
            pltpu.sync_copy(dh.at[pl.ds(base + ci * CH, CH)], dc0)
            pltpu.sync_copy(ones_v, dg.at[dc0], add=True)
            return c
        lax.fori_loop(0, CPT, chunk, 0)
    plsc.subcore_barrier()

    for r, dg in enumerate((deg0, deg1, deg2)):
        q = (cid * 3 + r) * N_ACC + row0
        pltpu.sync_copy(dg.at[pl.ds(row0, SLAB), :], out.at[pl.ds(q, SLAB), :])


def _hist_call(d0, d1, d2):
    f = pl.kernel(
        _hist_body,
        out_type=jax.ShapeDtypeStruct((NC * 3 * N_ACC, 16), jnp.float32),
        mesh=_mesh(),
        scratch_types=[
            pltpu.VMEM_SHARED((N_ACC, 16), jnp.float32),
            pltpu.VMEM_SHARED((N_ACC, 16), jnp.float32),
            pltpu.VMEM_SHARED((N_ACC, 16), jnp.float32),
            pltpu.VMEM((CH, 16), jnp.float32),
            pltpu.VMEM((SLAB, 16), jnp.float32),
            pltpu.VMEM((CH,), jnp.int32),
            pltpu.VMEM((CH,), jnp.int32),
            pltpu.VMEM((CH,), jnp.int32),
            pltpu.VMEM((CH,), jnp.int32),
            pltpu.SemaphoreType.DMA((4,)),
            pltpu.SemaphoreType.DMA((4,)),
        ],
    )
    return f(d0, d1, d2)


HC = CPT // 2


def _scatter_body(t0, t1, t2, s0_, d0_, s1_, d1_, s2_, d2_, out,
                  acc, src_all, dc0, dc1, rows, gsem, dsem):
    cid = lax.axis_index("c")
    sid = lax.axis_index("s")
    z16 = jnp.zeros((16,), jnp.float32)
    row0 = sid * SLAB
    base = (cid * NS + sid) * EPT
    dst2 = (dc0, dc1)

    def gath(tb, ci, j):
        pltpu.make_async_copy(tb.at[src_all.at[pl.ds(ci * CH, CH)]],
                              rows.at[j], gsem.at[j]).start()

    def gwait(tb, j):
        pltpu.make_async_copy(tb.at[src_all.at[pl.ds(0, CH)]],
                              rows.at[j], gsem.at[j]).wait()

    def iload(dh, ci, j):
        pltpu.make_async_copy(dh.at[pl.ds(base + ci * CH, CH)],
                              dst2[j], dsem.at[j]).start()

    def iwait(dh, j):
        pltpu.make_async_copy(dh.at[pl.ds(base, CH)],
                              dst2[j], dsem.at[j]).wait()

    for r, (tb, sh, dh) in enumerate(
            ((t0, s0_, d0_), (t1, s1_, d1_), (t2, s2_, d2_))):
        def init_z(k, c):
            rows[0, k // 8, pl.ds((k % 8) * 16, 16)] = z16
            return c
        lax.fori_loop(0, CH * 8, init_z, 0)
        for j in range(SLAB // CH):
            pltpu.sync_copy(rows.at[0], acc.at[pl.ds(row0 + j * CH, CH), :])
        plsc.subcore_barrier()

        for h in range(2):
            pltpu.sync_copy(sh.at[pl.ds(base + h * HC * CH, HC * CH)], src_all)
            for j in range(2):
                gath(tb, j, j)
                iload(dh, h * HC + j, j)

            def stage(i, c, tb=tb, dh=dh, h=h):
                c0 = 2 * i
                gwait(tb, 0)
                iwait(dh, 0)
                pltpu.sync_copy(rows.at[0], acc.at[dc0], add=True)
                gath(tb, c0 + 2, 0)
                iload(dh, h * HC + c0 + 2, 0)
                gwait(tb, 1)
                iwait(dh, 1)
                pltpu.sync_copy(rows.at[1], acc.at[dc1], add=True)
                gath(tb, c0 + 3, 1)
                iload(dh, h * HC + c0 + 3, 1)
                return c
            lax.fori_loop(0, HC // 2 - 1, stage, 0)
            for j in range(2):
                gwait(tb, j)
                iwait(dh, j)
                pltpu.sync_copy(rows.at[j], acc.at[dst2[j]], add=True)
        plsc.subcore_barrier()

        q = (r * NC + cid) * N_ACC + row0
        pltpu.sync_copy(acc.at[pl.ds(row0, SLAB), :], out.at[pl.ds(q, SLAB), :])


def _scatter_call(t0, t1, t2, s0, d0, s1, d1, s2, d2):
    f = pl.kernel(
        _scatter_body,
        out_type=jax.ShapeDtypeStruct((3 * NC * N_ACC, D), jnp.float32),
        mesh=_mesh(),
        scratch_types=[
            pltpu.VMEM_SHARED((N_ACC, D), jnp.float32),
            pltpu.VMEM((HC * CH,), jnp.int32),
            pltpu.VMEM((CH,), jnp.int32),
            pltpu.VMEM((CH,), jnp.int32),
            pltpu.VMEM((2, CH, D), jnp.float32),
            pltpu.SemaphoreType.DMA((2,)),
            pltpu.SemaphoreType.DMA((2,)),
        ],
    )
    return f(t0, t1, t2, s0, d0, s1, d1, s2, d2)


def _scale_body(xa, xs, dga, dgs, oa, os_):
    oa[...] = xa[...] * lax.rsqrt(dga[...])
    os_[...] = xs[...] * lax.rsqrt(dgs[...])


def _scale_call(x_article, x_software, deg_ref_col, deg_rel_col):
    nb = N // 1000
    row = pl.BlockSpec((1000, D), lambda i: (i, 0))
    col = pl.BlockSpec((1000, 1), lambda i: (i, 0))
    return pl.pallas_call(
        _scale_body,
        grid=(nb,),
        in_specs=[row, row, col, col],
        out_specs=[row, row],
        out_shape=[jax.ShapeDtypeStruct((N, D), jnp.float32)] * 2,
    )(x_article, x_software, deg_ref_col, deg_rel_col)


def _final_body(accs, xa, xs, dga, dgs, cnt, w_ref, w_rel, w_l, w_r,
                b_ref, b_rel, b_l, out_a, out_s):
    f32 = jnp.float32
    acc_ref = accs[0, 0] + accs[0, 1]
    acc_rel = accs[1, 0] + accs[1, 1]
    acc_men = accs[2, 0] + accs[2, 1]
    dinv_a = lax.rsqrt(dga[...])
    dinv_s = lax.rsqrt(dgs[...])
    xa_b = xa[...]
    xs_b = xs[...]
    gcn = jnp.dot(dinv_a * (acc_ref + dinv_a * xa_b), w_ref[...],
                  preferred_element_type=f32) + b_ref[...]
    mean = acc_men / jnp.maximum(cnt[...], 1.0)
    sage = (jnp.dot(mean, w_l[...], preferred_element_type=f32) + b_l[...]
            + jnp.dot(xa_b, w_r[...], preferred_element_type=f32))
    out_a[...] = jnp.maximum(0.5 * (gcn + sage), 0.0)
    gcn_s = jnp.dot(dinv_s * (acc_rel + dinv_s * xs_b), w_rel[...],
                    preferred_element_type=f32) + b_rel[...]
    out_s[...] = jnp.maximum(gcn_s, 0.0)


def _final_call(accs, x_article, x_software, deg_ref_col, deg_rel_col, cnt_col,
                w_ref, w_rel, w_l, w_r, b_ref, b_rel, b_l):
    nb = N // 1000
    row = pl.BlockSpec((1000, D), lambda i: (i, 0))
    col = pl.BlockSpec((1000, 1), lambda i: (i, 0))
    wsp = pl.BlockSpec((D, D), lambda i: (0, 0))
    bsp = pl.BlockSpec((1, D), lambda i: (0, 0))
    asp = pl.BlockSpec((3, NC, 1000, D), lambda i: (0, 0, i, 0))
    return pl.pallas_call(
        _final_body,
        grid=(nb,),
        in_specs=[asp, row, row, col, col, col, wsp, wsp, wsp, wsp,
                  bsp, bsp, bsp],
        out_specs=[row, row],
        out_shape=[jax.ShapeDtypeStruct((N, D), jnp.float32)] * 2,
    )(accs, x_article, x_software, deg_ref_col, deg_rel_col, cnt_col,
      w_ref, w_rel, w_l, w_r, b_ref, b_rel, b_l)


def kernel(x_article, x_software, edge_index_references, edge_index_related,
           edge_index_mentioned_in, W_gcn_ref, b_gcn_ref, W_gcn_rel, b_gcn_rel,
           W_sage_l, b_sage_l, W_sage_r):
    pad0 = jnp.zeros((E_PAD - E,), jnp.int32)
    padn = jnp.full((E_PAD - E,), N, jnp.int32)

    def prep(ei):
        s = jnp.concatenate([ei[0], pad0])
        d = jnp.concatenate([ei[1], padn])
        return s, d

    s_ref, d_ref = prep(edge_index_references)
    s_rel, d_rel = prep(edge_index_related)
    s_men, d_men = prep(edge_index_mentioned_in)

    deg_parts = _hist_call(d_ref, d_rel, d_men)
    dp = deg_parts.reshape(NC, 3, N_ACC, 16)[:, :, :N, 0]
    hist = dp[0] + dp[1]
    deg_ref_col = (hist[0] + 1.0).reshape(N, 1)
    deg_rel_col = (hist[1] + 1.0).reshape(N, 1)
    cnt_col = hist[2].reshape(N, 1)

    xs_ref, xs_rel = _scale_call(x_article, x_software, deg_ref_col, deg_rel_col)

    acc = _scatter_call(xs_ref, xs_rel, x_software,
                        s_ref, d_ref, s_rel, d_rel, s_men, d_men)
    accs = acc.reshape(3, NC, N_ACC, D)

    out_a, out_s = _final_call(
        accs, x_article, x_software, deg_ref_col, deg_rel_col, cnt_col,
        W_gcn_ref, W_gcn_rel, W_sage_l, W_sage_r,
        b_gcn_ref.reshape(1, D), b_gcn_rel.reshape(1, D), b_sage_l.reshape(1, D))
    return out_a, out_s
